# baseline (speedup 1.0000x reference)
import functools

import jax
import jax.numpy as jnp
from jax import lax
from jax.experimental import pallas as pl
from jax.experimental.pallas import tpu as pltpu
from jax.experimental.pallas import tpu_sc as plsc

_LANES = 16
_BLKW = 128


def _make_lookup(B, V, D):
    info = plsc.get_sparse_core_info()
    NC, NS = info.num_cores, info.num_subcores
    NW = NC * NS
    b_per_w = B // NW
    b_per_c = NS * b_per_w
    n_tc = b_per_c // _BLKW
    w_per_tc = _BLKW // b_per_w
    assert B % NW == 0 and b_per_w % _LANES == 0 and D == _LANES
    assert b_per_c % _BLKW == 0 and n_tc <= NS

    mesh = plsc.VectorSubcoreMesh(core_axis_name="c", subcore_axis_name="s")

    @functools.partial(
        pl.kernel,
        mesh=mesh,
        out_type=jax.ShapeDtypeStruct((D, B), jnp.float32),
        scratch_types=[
            pltpu.VMEM((b_per_w,), jnp.int32),
            pltpu.VMEM((b_per_w, D, _BLKW), jnp.float32),
            pltpu.VMEM((b_per_w * D,), jnp.float32),
            pltpu.VMEM_SHARED((b_per_c * D,), jnp.float32),
            pltpu.VMEM((w_per_tc * b_per_w * D,), jnp.float32),
            pltpu.VMEM((D, _BLKW), jnp.float32),
            pltpu.SemaphoreType.DMA,
        ],
        compiler_params=pltpu.CompilerParams(
            needs_layout_passes=False,
            disable_bounds_checks=True,
            skip_device_barrier=True,
        ),
    )
    def lookup(values_hbm, tab_t_hbm, out_hbm, idx_v, blks_v, out_v, stage_s,
               in_v, tile_v, sem):
        cid = lax.axis_index("c")
        sid = lax.axis_index("s")
        base = (cid * NS + sid) * b_per_w
        pltpu.sync_copy(values_hbm.at[pl.ds(base, b_per_w)], idx_v)

        lane = lax.iota(jnp.int32, _LANES)

        halves = [idx_v[pl.ds(h * _LANES, _LANES)] for h in range(b_per_w // _LANES)]
        starts = []
        vals = []
        for j in range(b_per_w):
            vj = halves[j // _LANES][j % _LANES]
            start = pl.multiple_of(
                lax.shift_left(lax.shift_right_logical(vj, 7), 7), 128
            )
            vals.append(vj)
            starts.append(start)

        copies = []
        for j in range(b_per_w):
            c = pltpu.async_copy(
                tab_t_hbm.at[:, pl.ds(starts[j], _BLKW)], blks_v.at[j], sem
            )
            copies.append(c)
        for c in copies:
            c.wait()

        for j in range(b_per_w):
            m = jnp.full((_LANES,), vals[j] - starts[j], jnp.int32)
            col = plsc.load_gather(blks_v, [jnp.full((_LANES,), j, jnp.int32), lane, m])
            plsc.store_scatter(out_v, [lane * b_per_w + j], col)

        pltpu.sync_copy(out_v, stage_s.at[pl.ds(sid * b_per_w * D, b_per_w * D)])
        plsc.subcore_barrier()

        @pl.when(sid < n_tc)
        def _write_tiles():
            pltpu.sync_copy(
                stage_s.at[pl.ds(sid * (w_per_tc * b_per_w * D), w_per_tc * b_per_w * D)],
                in_v,
            )
            for w in range(w_per_tc):
                for d in range(D):
                    for c8 in range(b_per_w // _LANES):
                        seg = in_v[
                            pl.ds(w * b_per_w * D + d * b_per_w + c8 * _LANES, _LANES)
                        ]
                        tile_v[d, pl.ds(w * b_per_w + c8 * _LANES, _LANES)] = seg
            col0 = pl.multiple_of((cid * n_tc + sid) * _BLKW, 128)
            pltpu.sync_copy(tile_v, out_hbm.at[:, pl.ds(col0, _BLKW)])

    return lookup


def kernel(values, table):
    B = values.shape[0]
    V, D = table.shape
    lookup = _make_lookup(B, V, D)
    out_t = lookup(values.astype(jnp.int32), table.T)
    return out_t.T


# speedup vs baseline: 1.0198x; 1.0198x over previous
import functools

import jax
import jax.numpy as jnp
from jax import lax
from jax.experimental import pallas as pl
from jax.experimental.pallas import tpu as pltpu
from jax.experimental.pallas import tpu_sc as plsc

_LANES = 16
_BLKW = 128


def _make_lookup(B, V, D):
    info = plsc.get_sparse_core_info()
    NC, NS = info.num_cores, info.num_subcores
    NW = NC * NS
    b_per_w = B // NW
    b_per_c = NS * b_per_w
    n_tc = b_per_c // _BLKW
    w_per_tc = _BLKW // b_per_w
    assert B % NW == 0 and b_per_w % _LANES == 0 and D == _LANES
    assert b_per_c % _BLKW == 0 and n_tc <= NS

    mesh = plsc.VectorSubcoreMesh(core_axis_name="c", subcore_axis_name="s")

    @functools.partial(
        pl.kernel,
        mesh=mesh,
        out_type=jax.ShapeDtypeStruct((D, B), jnp.float32),
        scratch_types=[
            pltpu.VMEM((b_per_w,), jnp.int32),
            pltpu.VMEM((b_per_w, D, _BLKW), jnp.float32),
            pltpu.VMEM((b_per_w * D,), jnp.float32),
            pltpu.VMEM_SHARED((b_per_c * D,), jnp.float32),
            pltpu.VMEM((w_per_tc * b_per_w * D,), jnp.float32),
            pltpu.VMEM((D, _BLKW), jnp.float32),
            pltpu.SemaphoreType.DMA,
        ],
        compiler_params=pltpu.CompilerParams(
            needs_layout_passes=False,
            disable_bounds_checks=True,
            skip_device_barrier=True,
        ),
    )
    def lookup(values_hbm, tab_t_hbm, out_hbm, idx_v, blks_v, out_v, stage_s,
               in_v, tile_v, sem):
        cid = lax.axis_index("c")
        sid = lax.axis_index("s")
        base = (cid * NS + sid) * b_per_w
        pltpu.sync_copy(values_hbm.at[pl.ds(base, b_per_w)], idx_v)

        lane = lax.iota(jnp.int32, _LANES)

        halves = [idx_v[pl.ds(h * _LANES, _LANES)] for h in range(b_per_w // _LANES)]
        starts = []
        vals = []
        for j in range(b_per_w):
            vj = halves[j // _LANES][j % _LANES]
            start = pl.multiple_of(
                lax.shift_left(lax.shift_right_logical(vj, 7), 7), 128
            )
            vals.append(vj)
            starts.append(start)

        copies = []
        for j in range(b_per_w):
            c = pltpu.async_copy(
                tab_t_hbm.at[:, pl.ds(starts[j], _BLKW)], blks_v.at[j], sem
            )
            copies.append(c)
        for j in range(b_per_w):
            copies[j].wait()
            m = jnp.full((_LANES,), vals[j] - starts[j], jnp.int32)
            col = plsc.load_gather(blks_v, [jnp.full((_LANES,), j, jnp.int32), lane, m])
            plsc.store_scatter(out_v, [lane * b_per_w + j], col)

        pltpu.sync_copy(out_v, stage_s.at[pl.ds(sid * b_per_w * D, b_per_w * D)])
        plsc.subcore_barrier()

        @pl.when(sid < n_tc)
        def _write_tiles():
            pltpu.sync_copy(
                stage_s.at[pl.ds(sid * (w_per_tc * b_per_w * D), w_per_tc * b_per_w * D)],
                in_v,
            )
            for w in range(w_per_tc):
                for d in range(D):
                    for c8 in range(b_per_w // _LANES):
                        seg = in_v[
                            pl.ds(w * b_per_w * D + d * b_per_w + c8 * _LANES, _LANES)
                        ]
                        tile_v[d, pl.ds(w * b_per_w + c8 * _LANES, _LANES)] = seg
            col0 = pl.multiple_of((cid * n_tc + sid) * _BLKW, 128)
            pltpu.sync_copy(tile_v, out_hbm.at[:, pl.ds(col0, _BLKW)])

    return lookup


def kernel(values, table):
    B = values.shape[0]
    V, D = table.shape
    lookup = _make_lookup(B, V, D)
    out_t = lookup(values.astype(jnp.int32), table.T)
    return out_t.T


# speedup vs baseline: 1.0276x; 1.0076x over previous
import functools

import jax
import jax.numpy as jnp
from jax import lax
from jax.experimental import pallas as pl
from jax.experimental.pallas import tpu as pltpu
from jax.experimental.pallas import tpu_sc as plsc

_LANES = 16
_BLKW = 128


def _make_lookup(B, V, D):
    info = plsc.get_sparse_core_info()
    NC, NS = info.num_cores, info.num_subcores
    NW = NC * NS
    b_per_w = B // NW
    b_per_c = NS * b_per_w
    n_tc = b_per_c // _BLKW
    w_per_tc = _BLKW // b_per_w
    assert B % NW == 0 and b_per_w % _LANES == 0 and D == _LANES
    assert b_per_c % _BLKW == 0 and n_tc <= NS

    mesh = plsc.VectorSubcoreMesh(core_axis_name="c", subcore_axis_name="s")

    @functools.partial(
        pl.kernel,
        mesh=mesh,
        out_type=jax.ShapeDtypeStruct((D, B), jnp.float32),
        scratch_types=[
            pltpu.VMEM((b_per_w,), jnp.int32),
            pltpu.VMEM((b_per_w,), jnp.int32),
            pltpu.VMEM((b_per_w,), jnp.int32),
            pltpu.VMEM((b_per_w, D, _BLKW), jnp.float32),
            pltpu.VMEM((b_per_w * D,), jnp.float32),
            pltpu.VMEM_SHARED((b_per_c * D,), jnp.float32),
            pltpu.VMEM((w_per_tc * b_per_w * D,), jnp.float32),
            pltpu.VMEM((D, _BLKW), jnp.float32),
            pltpu.SemaphoreType.DMA,
        ],
        compiler_params=pltpu.CompilerParams(
            needs_layout_passes=False,
            disable_bounds_checks=True,
            skip_device_barrier=True,
        ),
    )
    def lookup(values_hbm, tab_t_hbm, out_hbm, idx_v, st_v, mo_v, blks_v,
               out_v, stage_s, in_v, tile_v, sem):
        cid = lax.axis_index("c")
        sid = lax.axis_index("s")
        base = (cid * NS + sid) * b_per_w
        pltpu.sync_copy(values_hbm.at[pl.ds(base, b_per_w)], idx_v)

        lane = lax.iota(jnp.int32, _LANES)

        for h in range(b_per_w // _LANES):
            vv = idx_v[pl.ds(h * _LANES, _LANES)]
            st_v[pl.ds(h * _LANES, _LANES)] = lax.shift_left(
                lax.shift_right_logical(vv, 7), 7
            )
            mo_v[pl.ds(h * _LANES, _LANES)] = lax.bitwise_and(vv, 127)

        def fire(j, _):
            jv = jnp.full((_LANES,), j, jnp.int32)
            start = pl.multiple_of(plsc.load_gather(st_v, [jv])[0], 128)
            pltpu.async_copy(
                tab_t_hbm.at[:, pl.ds(start, _BLKW)], blks_v.at[j], sem
            )
            return 0

        lax.fori_loop(0, b_per_w, fire, 0)

        def drain_extract(j, _):
            pltpu.make_async_copy(
                tab_t_hbm.at[:, pl.ds(0, _BLKW)], blks_v.at[j], sem
            ).wait()
            jv = jnp.full((_LANES,), j, jnp.int32)
            m = plsc.load_gather(mo_v, [jv])
            col = plsc.load_gather(blks_v, [jv, lane, m])
            plsc.store_scatter(out_v, [lane * b_per_w + j], col)
            return 0

        lax.fori_loop(0, b_per_w, drain_extract, 0)

        pltpu.sync_copy(out_v, stage_s.at[pl.ds(sid * b_per_w * D, b_per_w * D)])
        plsc.subcore_barrier()

        @pl.when(sid < n_tc)
        def _write_tiles():
            pltpu.sync_copy(
                stage_s.at[pl.ds(sid * (w_per_tc * b_per_w * D), w_per_tc * b_per_w * D)],
                in_v,
            )

            def assemble(w, _):
                for d in range(D):
                    for c8 in range(b_per_w // _LANES):
                        seg = in_v[
                            pl.ds(w * (b_per_w * D) + d * b_per_w + c8 * _LANES, _LANES)
                        ]
                        tile_v[d, pl.ds(w * b_per_w + c8 * _LANES, _LANES)] = seg
                return 0

            lax.fori_loop(0, w_per_tc, assemble, 0)
            col0 = pl.multiple_of((cid * n_tc + sid) * _BLKW, 128)
            pltpu.sync_copy(tile_v, out_hbm.at[:, pl.ds(col0, _BLKW)])

    return lookup


def kernel(values, table):
    B = values.shape[0]
    V, D = table.shape
    lookup = _make_lookup(B, V, D)
    out_t = lookup(values.astype(jnp.int32), table.T)
    return out_t.T
